# baseline (speedup 1.0000x reference)
import functools

import jax
import jax.numpy as jnp
from jax import lax
from jax.experimental import pallas as pl
from jax.experimental.pallas import tpu as pltpu
from jax.experimental.pallas import tpu_sc as plsc

N = 10000
E = 320000
D_IN = 8192
D_H = 200
DP = 208
NC = 2
NS = 16
NW = NC * NS
CH = 128
NCHUNK = E // CH
ROWS_PER_TILE = N // NS
WB = 125
BR = 400
BK = 2048


def _sc_mesh():
    return plsc.VectorSubcoreMesh(
        core_axis_name="c", subcore_axis_name="s", num_cores=NC, num_subcores=NS
    )


def _n_chunks_for(w):
    return jnp.where(w < NCHUNK - (NCHUNK // NW) * NW, NCHUNK // NW + 1, NCHUNK // NW)


def _sc_degree(dst):
    @functools.partial(
        pl.kernel,
        out_type=jax.ShapeDtypeStruct((NC, N, 16), jnp.float32),
        mesh=_sc_mesh(),
        scratch_types=[
            pltpu.VMEM((1, CH), jnp.int32),
            pltpu.VMEM((CH, 16), jnp.float32),
            pltpu.VMEM((ROWS_PER_TILE, 16), jnp.float32),
            pltpu.VMEM_SHARED((N, 16), jnp.float32),
        ],
        compiler_params=pltpu.CompilerParams(use_tc_tiling_on_sc=False),
    )
    def k(dst_hbm, out_hbm, idx_v, ones_v, wb_v, deg_sh):
        c = lax.axis_index("c")
        s = lax.axis_index("s")
        w = s * NC + c

        def fill_ones(i, carry):
            ones_v[i, :] = jnp.full((16,), 1.0, jnp.float32)
            return carry

        lax.fori_loop(0, CH, fill_ones, 0)

        def fill_zero(i, carry):
            wb_v[i, :] = jnp.zeros((16,), jnp.float32)
            return carry

        lax.fori_loop(0, ROWS_PER_TILE, fill_zero, 0)
        pltpu.sync_copy(wb_v, deg_sh.at[pl.ds(s * ROWS_PER_TILE, ROWS_PER_TILE)])
        plsc.subcore_barrier()

        def body(j, carry):
            chunk = w + j * NW
            pltpu.sync_copy(dst_hbm.at[pl.ds(chunk * CH, CH)], idx_v.at[0])
            pltpu.sync_copy(ones_v, deg_sh.at[idx_v.at[0]], add=True)
            return carry

        lax.fori_loop(0, _n_chunks_for(w), body, 0)
        plsc.subcore_barrier()
        pltpu.sync_copy(deg_sh.at[pl.ds(s * ROWS_PER_TILE, ROWS_PER_TILE)], wb_v)
        pltpu.sync_copy(wb_v, out_hbm.at[c, pl.ds(s * ROWS_PER_TILE, ROWS_PER_TILE)])

    return k(dst)


HALF = N // NC
ACC_ROWS = 5040
ZCH = 40
IB = 8
IDX_E = IB * CH
BLKS = 20
CPS = BLKS * IB
EPAD = NS * CPS * CH


def _split_count(n_items, s):
    q, r = n_items // NS, n_items % NS
    return jnp.where(s < r, q + 1, q)


def _sc_aggregate(g, src, dst):
    CB = 48
    CAP = CPS * CH + 2 * CB
    TRASH_P = HALF

    @functools.partial(
        pl.kernel,
        out_type=jax.ShapeDtypeStruct((N, DP), jnp.float32),
        mesh=_sc_mesh(),
        scratch_types=[
            pltpu.VMEM((IDX_E,), jnp.int32),
            pltpu.VMEM((IDX_E,), jnp.int32),
            pltpu.VMEM((IDX_E,), jnp.int32),
            pltpu.VMEM((IDX_E,), jnp.int32),
            pltpu.VMEM((CAP,), jnp.int32),
            pltpu.VMEM((1, CB), jnp.int32),
            pltpu.VMEM((1, CB), jnp.int32),
            pltpu.VMEM((1, CB), jnp.int32),
            pltpu.VMEM((1, CB), jnp.int32),
            pltpu.VMEM((CB, DP), jnp.float32),
            pltpu.VMEM((CB, DP), jnp.float32),
            pltpu.SemaphoreType.DMA,
            pltpu.SemaphoreType.DMA,
            pltpu.SemaphoreType.DMA,
            pltpu.SemaphoreType.DMA,
            pltpu.SemaphoreType.DMA,
            pltpu.SemaphoreType.DMA,
            pltpu.VMEM_SHARED((ACC_ROWS, DP), jnp.float32),
        ],
        compiler_params=pltpu.CompilerParams(
            use_tc_tiling_on_sc=False, needs_layout_passes=False
        ),
    )
    def k(g_hbm, src_hbm, dst_hbm, out_hbm,
          sblk0, sblk1, dblk0, dblk1, cpk, sb0, sb1, db0, db1, rows0, rows1,
          isem0, isem1, gsem0, gsem1, ssem0, ssem1, agg_sh):
        c = lax.axis_index("c")
        s = lax.axis_index("s")
        base = c * HALF
        sblk = (sblk0, sblk1)
        dblk = (dblk0, dblk1)
        sb = (sb0, sb1)
        db = (db0, db1)
        rows = (rows0, rows1)
        isem = (isem0, isem1)
        gsem = (gsem0, gsem1)
        ssem = (ssem0, ssem1)

        def zrow(i, carry):
            def zcol(jj, inner):
                rows0[i, pl.ds(jj * 16, 16)] = jnp.zeros((16,), jnp.float32)
                return inner

            lax.fori_loop(0, DP // 16, zcol, 0)
            return carry

        lax.fori_loop(0, ZCH, zrow, 0)

        def zchunk(t, carry):
            pltpu.sync_copy(rows0.at[pl.ds(0, ZCH)],
                            agg_sh.at[pl.ds((s + t * NS) * ZCH, ZCH)])
            return carry

        lax.fori_loop(0, _split_count(ACC_ROWS // ZCH, s), zchunk, 0)
        plsc.subcore_barrier()

        def load_blk(t, b):
            off = s * (CPS * CH) + t * IDX_E
            pltpu.async_copy(src_hbm.at[pl.ds(off, IDX_E)], sblk[b], isem[b])
            pltpu.async_copy(dst_hbm.at[pl.ds(off, IDX_E)], dblk[b], isem[b])

        def wait_blk(b):
            pltpu.make_async_copy(src_hbm.at[pl.ds(0, IDX_E)], sblk[b],
                                  isem[b]).wait()
            pltpu.make_async_copy(src_hbm.at[pl.ds(0, IDX_E)], dblk[b],
                                  isem[b]).wait()

        def compact_blk(b, fill):
            for v in range(IDX_E // 16):
                d = dblk[b][pl.ds(v * 16, 16)]
                sv = sblk[b][pl.ds(v * 16, 16)]
                ok = (d >= base) & (d < base + HALF)
                p = sv * 8192 + (d - base)
                _, p_sorted = plsc.sort_key_val(1 - ok.astype(jnp.int32), p)
                cpk[pl.ds(fill, 16)] = p_sorted
                fill = fill + plsc.all_reduce_population_count(ok)[0]
            return fill

        def phase_a(t, b, fill):
            wait_blk(b)
            fill = compact_blk(b, fill)

            @pl.when(t + 2 < BLKS)
            def _():
                load_blk(t + 2, b)

            return fill

        load_blk(0, 0)
        load_blk(1, 1)

        def a_pair(t, fill):
            fill = phase_a(2 * t, 0, fill)
            fill = phase_a(2 * t + 1, 1, fill)
            return fill

        fill = lax.fori_loop(0, BLKS // 2, a_pair, jnp.int32(0))

        trash = jnp.full((16,), TRASH_P, jnp.int32)
        for q in range(2 * CB // 16):
            cpk[pl.ds(fill + 16 * q, 16)] = trash
        nbat = (fill // (2 * CB) + 1) * 2

        def unpack(t, b):
            for v in range(CB // 16):
                p = cpk[pl.ds(t * CB + v * 16, 16)]
                sb[b][0, pl.ds(v * 16, 16)] = lax.shift_right_logical(p, 13)
                db[b][0, pl.ds(v * 16, 16)] = lax.bitwise_and(p, 8191)

        def start_gather(b):
            pltpu.async_copy(g_hbm.at[sb[b].at[0]], rows[b], gsem[b])

        def wait_gather(b):
            pltpu.make_async_copy(g_hbm.at[pl.ds(0, CB)], rows[b],
                                  gsem[b]).wait()

        def start_scatter(b):
            pltpu.async_copy(rows[b], agg_sh.at[db[b].at[0]], ssem[b],
                             add=True)

        def wait_scatter(b):
            pltpu.make_async_copy(g_hbm.at[pl.ds(0, CB)], rows[b],
                                  ssem[b]).wait()

        unpack(0, 0)
        start_gather(0)
        unpack(1, 1)
        start_gather(1)
        wait_gather(0)
        start_scatter(0)

        def b_step(t, a, b):
            @pl.when(t < nbat)
            def _():
                wait_scatter(a)
                unpack(t, a)
                start_gather(a)
                wait_gather(b)
                start_scatter(b)

        def b_pair(t, carry):
            b_step(2 + 2 * t, 0, 1)
            b_step(3 + 2 * t, 1, 0)
            return carry

        lax.fori_loop(0, (nbat - 2) // 2, b_pair, 0)
        wait_gather(1)
        start_scatter(1)
        wait_scatter(0)
        wait_scatter(1)
        plsc.subcore_barrier()

        n_wb = _split_count(HALF // ZCH, s)

        def wbchunk(t, carry):
            r0 = (s + t * NS) * ZCH
            pltpu.sync_copy(agg_sh.at[pl.ds(r0, ZCH)],
                            out_hbm.at[pl.ds(base + r0, ZCH)])
            return carry

        lax.fori_loop(0, n_wb, wbchunk, 0)

    return k(g, src, dst)


def _tc_encode(x, w_in, b_in, w1):
    n_r, n_k = N // BR, D_IN // BK

    def body(x_ref, w_ref, b_ref, w1_ref, hw_ref, acc_ref):
        kk = pl.program_id(1)

        @pl.when(kk == 0)
        def _():
            acc_ref[...] = jnp.zeros_like(acc_ref)

        acc_ref[...] += jnp.dot(
            x_ref[...], w_ref[...], preferred_element_type=jnp.float32
        )

        @pl.when(kk == n_k - 1)
        def _():
            h0 = jnp.maximum(acc_ref[...] + b_ref[...], 0.0)
            hw_ref[...] = jnp.dot(
                h0, w1_ref[...], preferred_element_type=jnp.float32
            )

    return pl.pallas_call(
        body,
        grid=(n_r, n_k),
        in_specs=[
            pl.BlockSpec((BR, BK), lambda i, k: (i, k)),
            pl.BlockSpec((BK, DP), lambda i, k: (k, 0)),
            pl.BlockSpec((1, DP), lambda i, k: (0, 0)),
            pl.BlockSpec((DP, DP), lambda i, k: (0, 0)),
        ],
        out_specs=pl.BlockSpec((BR, DP), lambda i, k: (i, 0)),
        out_shape=jax.ShapeDtypeStruct((N, DP), jnp.float32),
        scratch_shapes=[pltpu.VMEM((BR, DP), jnp.float32)],
        compiler_params=pltpu.CompilerParams(
            dimension_semantics=("parallel", "arbitrary")
        ),
    )(x, w_in, b_in, w1)


def _tc_prep(hw, deg0, deg1):
    def body(hw_ref, d0_ref, d1_ref, g_ref, dinv_ref):
        deg = d0_ref[:, 0:1] + d1_ref[:, 0:1] + 2.0
        dinv = lax.rsqrt(deg)
        g_ref[...] = dinv * hw_ref[...]
        dinv_ref[...] = dinv

    return pl.pallas_call(
        body,
        grid=(N // BR,),
        in_specs=[
            pl.BlockSpec((BR, DP), lambda i: (i, 0)),
            pl.BlockSpec((BR, 16), lambda i: (i, 0)),
            pl.BlockSpec((BR, 16), lambda i: (i, 0)),
        ],
        out_specs=[
            pl.BlockSpec((BR, DP), lambda i: (i, 0)),
            pl.BlockSpec((BR, 1), lambda i: (i, 0)),
        ],
        out_shape=[
            jax.ShapeDtypeStruct((N, DP), jnp.float32),
            jax.ShapeDtypeStruct((N, 1), jnp.float32),
        ],
    )(hw, deg0, deg1)


def _tc_combine(s_agg, hw, dinv, b, w_next):
    def body(s_ref, hw_ref, dinv_ref, b_ref, w_ref, g_ref, hwn_ref):
        dv = dinv_ref[...]
        h = dv * s_ref[...] + (2.0 * dv * dv) * hw_ref[...] + b_ref[...]
        hwn = jnp.dot(h, w_ref[...], preferred_element_type=jnp.float32)
        hwn_ref[...] = hwn
        g_ref[...] = dv * hwn

    return pl.pallas_call(
        body,
        grid=(N // BR,),
        in_specs=[
            pl.BlockSpec((BR, DP), lambda i: (i, 0)),
            pl.BlockSpec((BR, DP), lambda i: (i, 0)),
            pl.BlockSpec((BR, 1), lambda i: (i, 0)),
            pl.BlockSpec((1, DP), lambda i: (0, 0)),
            pl.BlockSpec((DP, DP), lambda i: (0, 0)),
        ],
        out_specs=[
            pl.BlockSpec((BR, DP), lambda i: (i, 0)),
            pl.BlockSpec((BR, DP), lambda i: (i, 0)),
        ],
        out_shape=[
            jax.ShapeDtypeStruct((N, DP), jnp.float32),
            jax.ShapeDtypeStruct((N, DP), jnp.float32),
        ],
    )(s_agg, hw, dinv, b, w_next)


def _tc_head(s_agg, hw, dinv, b, w_out_row, b_out):
    def body(s_ref, hw_ref, dinv_ref, b_ref, w_ref, bo_ref, o_ref):
        dv = dinv_ref[...]
        h = dv * s_ref[...] + (2.0 * dv * dv) * hw_ref[...] + b_ref[...]
        z = jnp.sum(h * w_ref[...], axis=1, keepdims=True) + bo_ref[0, 0]
        o_ref[...] = jax.nn.sigmoid(z)

    return pl.pallas_call(
        body,
        grid=(N // BR,),
        in_specs=[
            pl.BlockSpec((BR, DP), lambda i: (i, 0)),
            pl.BlockSpec((BR, DP), lambda i: (i, 0)),
            pl.BlockSpec((BR, 1), lambda i: (i, 0)),
            pl.BlockSpec((1, DP), lambda i: (0, 0)),
            pl.BlockSpec((1, DP), lambda i: (0, 0)),
            pl.BlockSpec((1, 1), lambda i: (0, 0)),
        ],
        out_specs=pl.BlockSpec((BR, 1), lambda i: (i, 0)),
        out_shape=jax.ShapeDtypeStruct((N, 1), jnp.float32),
    )(s_agg, hw, dinv, b, w_out_row, b_out)


def _pad_w(w):
    return jnp.pad(w, ((0, 0), (0, DP - w.shape[1])))


def _pad_sq(w):
    return jnp.pad(w, ((0, DP - w.shape[0]), (0, DP - w.shape[1])))


def _pad_b(b):
    return jnp.pad(b, (0, DP - b.shape[0])).reshape(1, DP)


def kernel(x, edge_index, W_in, b_in, W1, b1, W2, b2, W3, b3, W_out, b_out):
    src = edge_index[0]
    dst = edge_index[1]

    w_in_p = _pad_w(W_in)
    w1_p = _pad_sq(W1)
    w2_p = _pad_sq(W2)
    w3_p = _pad_sq(W3)
    b_in_p = _pad_b(b_in)
    b1_p = _pad_b(b1)
    b2_p = _pad_b(b2)
    b3_p = _pad_b(b3)
    w_out_row = jnp.pad(W_out[:, 0], (0, DP - D_H)).reshape(1, DP)
    b_out_2d = b_out.reshape(1, 1)

    src_p = jnp.pad(src, (0, EPAD - E))
    dst_p = jnp.pad(dst, (0, EPAD - E), constant_values=N)

    degp = _sc_degree(dst)
    hw1 = _tc_encode(x, w_in_p, b_in_p, w1_p)
    g1, dinv = _tc_prep(hw1, degp[0], degp[1])
    s1 = _sc_aggregate(g1, src_p, dst_p)
    g2, hw2 = _tc_combine(s1, hw1, dinv, b1_p, w2_p)
    s2 = _sc_aggregate(g2, src_p, dst_p)
    g3, hw3 = _tc_combine(s2, hw2, dinv, b2_p, w3_p)
    s3 = _sc_aggregate(g3, src_p, dst_p)
    out = _tc_head(s3, hw3, dinv, b3_p, w_out_row, b_out_2d)
    return out.reshape(1, -1)


# speedup vs baseline: 1.0019x; 1.0019x over previous
import functools

import jax
import jax.numpy as jnp
from jax import lax
from jax.experimental import pallas as pl
from jax.experimental.pallas import tpu as pltpu
from jax.experimental.pallas import tpu_sc as plsc

N = 10000
E = 320000
D_IN = 8192
D_H = 200
DP = 208
NC = 2
NS = 16
NW = NC * NS
CH = 128
NCHUNK = E // CH
ROWS_PER_TILE = N // NS
WB = 125
BR = 400
BK = 2048


def _sc_mesh():
    return plsc.VectorSubcoreMesh(
        core_axis_name="c", subcore_axis_name="s", num_cores=NC, num_subcores=NS
    )


def _n_chunks_for(w):
    return jnp.where(w < NCHUNK - (NCHUNK // NW) * NW, NCHUNK // NW + 1, NCHUNK // NW)


def _sc_degree(dst):
    @functools.partial(
        pl.kernel,
        out_type=jax.ShapeDtypeStruct((NC, N, 16), jnp.float32),
        mesh=_sc_mesh(),
        scratch_types=[
            pltpu.VMEM((1, CH), jnp.int32),
            pltpu.VMEM((CH, 16), jnp.float32),
            pltpu.VMEM((ROWS_PER_TILE, 16), jnp.float32),
            pltpu.VMEM_SHARED((N, 16), jnp.float32),
        ],
        compiler_params=pltpu.CompilerParams(use_tc_tiling_on_sc=False),
    )
    def k(dst_hbm, out_hbm, idx_v, ones_v, wb_v, deg_sh):
        c = lax.axis_index("c")
        s = lax.axis_index("s")
        w = s * NC + c

        def fill_ones(i, carry):
            ones_v[i, :] = jnp.full((16,), 1.0, jnp.float32)
            return carry

        lax.fori_loop(0, CH, fill_ones, 0)

        def fill_zero(i, carry):
            wb_v[i, :] = jnp.zeros((16,), jnp.float32)
            return carry

        lax.fori_loop(0, ROWS_PER_TILE, fill_zero, 0)
        pltpu.sync_copy(wb_v, deg_sh.at[pl.ds(s * ROWS_PER_TILE, ROWS_PER_TILE)])
        plsc.subcore_barrier()

        def body(j, carry):
            chunk = w + j * NW
            pltpu.sync_copy(dst_hbm.at[pl.ds(chunk * CH, CH)], idx_v.at[0])
            pltpu.sync_copy(ones_v, deg_sh.at[idx_v.at[0]], add=True)
            return carry

        lax.fori_loop(0, _n_chunks_for(w), body, 0)
        plsc.subcore_barrier()
        pltpu.sync_copy(deg_sh.at[pl.ds(s * ROWS_PER_TILE, ROWS_PER_TILE)], wb_v)
        pltpu.sync_copy(wb_v, out_hbm.at[c, pl.ds(s * ROWS_PER_TILE, ROWS_PER_TILE)])

    return k(dst)


HALF = N // NC
ACC_ROWS = 5040
ZCH = 40
IB = 8
IDX_E = IB * CH
BLKS = 20
CPS = BLKS * IB
EPAD = NS * CPS * CH


def _split_count(n_items, s):
    q, r = n_items // NS, n_items % NS
    return jnp.where(s < r, q + 1, q)


def _sc_aggregate(g, src, dst):
    CB = 64
    CAP = CPS * CH + 2 * CB
    TRASH_P = HALF

    @functools.partial(
        pl.kernel,
        out_type=jax.ShapeDtypeStruct((N, DP), jnp.float32),
        mesh=_sc_mesh(),
        scratch_types=[
            pltpu.VMEM((IDX_E,), jnp.int32),
            pltpu.VMEM((IDX_E,), jnp.int32),
            pltpu.VMEM((IDX_E,), jnp.int32),
            pltpu.VMEM((IDX_E,), jnp.int32),
            pltpu.VMEM((CAP,), jnp.int32),
            pltpu.VMEM((1, CB), jnp.int32),
            pltpu.VMEM((1, CB), jnp.int32),
            pltpu.VMEM((1, CB), jnp.int32),
            pltpu.VMEM((1, CB), jnp.int32),
            pltpu.VMEM((CB, DP), jnp.float32),
            pltpu.VMEM((CB, DP), jnp.float32),
            pltpu.SemaphoreType.DMA,
            pltpu.SemaphoreType.DMA,
            pltpu.SemaphoreType.DMA,
            pltpu.SemaphoreType.DMA,
            pltpu.SemaphoreType.DMA,
            pltpu.SemaphoreType.DMA,
            pltpu.VMEM_SHARED((ACC_ROWS, DP), jnp.float32),
        ],
        compiler_params=pltpu.CompilerParams(
            use_tc_tiling_on_sc=False, needs_layout_passes=False
        ),
    )
    def k(g_hbm, src_hbm, dst_hbm, out_hbm,
          sblk0, sblk1, dblk0, dblk1, cpk, sb0, sb1, db0, db1, rows0, rows1,
          isem0, isem1, gsem0, gsem1, ssem0, ssem1, agg_sh):
        c = lax.axis_index("c")
        s = lax.axis_index("s")
        base = c * HALF
        sblk = (sblk0, sblk1)
        dblk = (dblk0, dblk1)
        sb = (sb0, sb1)
        db = (db0, db1)
        rows = (rows0, rows1)
        isem = (isem0, isem1)
        gsem = (gsem0, gsem1)
        ssem = (ssem0, ssem1)

        def zrow(i, carry):
            def zcol(jj, inner):
                rows0[i, pl.ds(jj * 16, 16)] = jnp.zeros((16,), jnp.float32)
                return inner

            lax.fori_loop(0, DP // 16, zcol, 0)
            return carry

        lax.fori_loop(0, ZCH, zrow, 0)

        def zchunk(t, carry):
            pltpu.sync_copy(rows0.at[pl.ds(0, ZCH)],
                            agg_sh.at[pl.ds((s + t * NS) * ZCH, ZCH)])
            return carry

        lax.fori_loop(0, _split_count(ACC_ROWS // ZCH, s), zchunk, 0)
        plsc.subcore_barrier()

        def load_blk(t, b):
            off = s * (CPS * CH) + t * IDX_E
            pltpu.async_copy(src_hbm.at[pl.ds(off, IDX_E)], sblk[b], isem[b])
            pltpu.async_copy(dst_hbm.at[pl.ds(off, IDX_E)], dblk[b], isem[b])

        def wait_blk(b):
            pltpu.make_async_copy(src_hbm.at[pl.ds(0, IDX_E)], sblk[b],
                                  isem[b]).wait()
            pltpu.make_async_copy(src_hbm.at[pl.ds(0, IDX_E)], dblk[b],
                                  isem[b]).wait()

        def compact_blk(b, fill):
            for v in range(IDX_E // 16):
                d = dblk[b][pl.ds(v * 16, 16)]
                sv = sblk[b][pl.ds(v * 16, 16)]
                ok = (d >= base) & (d < base + HALF)
                p = sv * 8192 + (d - base)
                _, p_sorted = plsc.sort_key_val(1 - ok.astype(jnp.int32), p)
                cpk[pl.ds(fill, 16)] = p_sorted
                fill = fill + plsc.all_reduce_population_count(ok)[0]
            return fill

        def phase_a(t, b, fill):
            wait_blk(b)
            fill = compact_blk(b, fill)

            @pl.when(t + 2 < BLKS)
            def _():
                load_blk(t + 2, b)

            return fill

        load_blk(0, 0)
        load_blk(1, 1)

        def a_pair(t, fill):
            fill = phase_a(2 * t, 0, fill)
            fill = phase_a(2 * t + 1, 1, fill)
            return fill

        fill = lax.fori_loop(0, BLKS // 2, a_pair, jnp.int32(0))

        trash = jnp.full((16,), TRASH_P, jnp.int32)
        for q in range(2 * CB // 16):
            cpk[pl.ds(fill + 16 * q, 16)] = trash
        nbat = (fill // (2 * CB) + 1) * 2

        def unpack(t, b):
            for v in range(CB // 16):
                p = cpk[pl.ds(t * CB + v * 16, 16)]
                sb[b][0, pl.ds(v * 16, 16)] = lax.shift_right_logical(p, 13)
                db[b][0, pl.ds(v * 16, 16)] = lax.bitwise_and(p, 8191)

        def start_gather(b):
            pltpu.async_copy(g_hbm.at[sb[b].at[0]], rows[b], gsem[b])

        def wait_gather(b):
            pltpu.make_async_copy(g_hbm.at[pl.ds(0, CB)], rows[b],
                                  gsem[b]).wait()

        def start_scatter(b):
            pltpu.async_copy(rows[b], agg_sh.at[db[b].at[0]], ssem[b],
                             add=True)

        def wait_scatter(b):
            pltpu.make_async_copy(g_hbm.at[pl.ds(0, CB)], rows[b],
                                  ssem[b]).wait()

        unpack(0, 0)
        start_gather(0)
        unpack(1, 1)
        start_gather(1)
        wait_gather(0)
        start_scatter(0)

        def b_step(t, a, b):
            @pl.when(t < nbat)
            def _():
                wait_scatter(a)
                unpack(t, a)
                start_gather(a)
                wait_gather(b)
                start_scatter(b)

        def b_pair(t, carry):
            b_step(2 + 2 * t, 0, 1)
            b_step(3 + 2 * t, 1, 0)
            return carry

        lax.fori_loop(0, (nbat - 2) // 2, b_pair, 0)
        wait_gather(1)
        start_scatter(1)
        wait_scatter(0)
        wait_scatter(1)
        plsc.subcore_barrier()

        n_wb = _split_count(HALF // ZCH, s)

        def wbchunk(t, carry):
            r0 = (s + t * NS) * ZCH
            pltpu.sync_copy(agg_sh.at[pl.ds(r0, ZCH)],
                            out_hbm.at[pl.ds(base + r0, ZCH)])
            return carry

        lax.fori_loop(0, n_wb, wbchunk, 0)

    return k(g, src, dst)


def _tc_encode(x, w_in, b_in, w1):
    n_r, n_k = N // BR, D_IN // BK

    def body(x_ref, w_ref, b_ref, w1_ref, hw_ref, acc_ref):
        kk = pl.program_id(1)

        @pl.when(kk == 0)
        def _():
            acc_ref[...] = jnp.zeros_like(acc_ref)

        acc_ref[...] += jnp.dot(
            x_ref[...], w_ref[...], preferred_element_type=jnp.float32
        )

        @pl.when(kk == n_k - 1)
        def _():
            h0 = jnp.maximum(acc_ref[...] + b_ref[...], 0.0)
            hw_ref[...] = jnp.dot(
                h0, w1_ref[...], preferred_element_type=jnp.float32
            )

    return pl.pallas_call(
        body,
        grid=(n_r, n_k),
        in_specs=[
            pl.BlockSpec((BR, BK), lambda i, k: (i, k)),
            pl.BlockSpec((BK, DP), lambda i, k: (k, 0)),
            pl.BlockSpec((1, DP), lambda i, k: (0, 0)),
            pl.BlockSpec((DP, DP), lambda i, k: (0, 0)),
        ],
        out_specs=pl.BlockSpec((BR, DP), lambda i, k: (i, 0)),
        out_shape=jax.ShapeDtypeStruct((N, DP), jnp.float32),
        scratch_shapes=[pltpu.VMEM((BR, DP), jnp.float32)],
        compiler_params=pltpu.CompilerParams(
            dimension_semantics=("parallel", "arbitrary")
        ),
    )(x, w_in, b_in, w1)


def _tc_prep(hw, deg0, deg1):
    def body(hw_ref, d0_ref, d1_ref, g_ref, dinv_ref):
        deg = d0_ref[:, 0:1] + d1_ref[:, 0:1] + 2.0
        dinv = lax.rsqrt(deg)
        g_ref[...] = dinv * hw_ref[...]
        dinv_ref[...] = dinv

    return pl.pallas_call(
        body,
        grid=(N // BR,),
        in_specs=[
            pl.BlockSpec((BR, DP), lambda i: (i, 0)),
            pl.BlockSpec((BR, 16), lambda i: (i, 0)),
            pl.BlockSpec((BR, 16), lambda i: (i, 0)),
        ],
        out_specs=[
            pl.BlockSpec((BR, DP), lambda i: (i, 0)),
            pl.BlockSpec((BR, 1), lambda i: (i, 0)),
        ],
        out_shape=[
            jax.ShapeDtypeStruct((N, DP), jnp.float32),
            jax.ShapeDtypeStruct((N, 1), jnp.float32),
        ],
    )(hw, deg0, deg1)


def _tc_combine(s_agg, hw, dinv, b, w_next):
    def body(s_ref, hw_ref, dinv_ref, b_ref, w_ref, g_ref, hwn_ref):
        dv = dinv_ref[...]
        h = dv * s_ref[...] + (2.0 * dv * dv) * hw_ref[...] + b_ref[...]
        hwn = jnp.dot(h, w_ref[...], preferred_element_type=jnp.float32)
        hwn_ref[...] = hwn
        g_ref[...] = dv * hwn

    return pl.pallas_call(
        body,
        grid=(N // BR,),
        in_specs=[
            pl.BlockSpec((BR, DP), lambda i: (i, 0)),
            pl.BlockSpec((BR, DP), lambda i: (i, 0)),
            pl.BlockSpec((BR, 1), lambda i: (i, 0)),
            pl.BlockSpec((1, DP), lambda i: (0, 0)),
            pl.BlockSpec((DP, DP), lambda i: (0, 0)),
        ],
        out_specs=[
            pl.BlockSpec((BR, DP), lambda i: (i, 0)),
            pl.BlockSpec((BR, DP), lambda i: (i, 0)),
        ],
        out_shape=[
            jax.ShapeDtypeStruct((N, DP), jnp.float32),
            jax.ShapeDtypeStruct((N, DP), jnp.float32),
        ],
    )(s_agg, hw, dinv, b, w_next)


def _tc_head(s_agg, hw, dinv, b, w_out_row, b_out):
    def body(s_ref, hw_ref, dinv_ref, b_ref, w_ref, bo_ref, o_ref):
        dv = dinv_ref[...]
        h = dv * s_ref[...] + (2.0 * dv * dv) * hw_ref[...] + b_ref[...]
        z = jnp.sum(h * w_ref[...], axis=1, keepdims=True) + bo_ref[0, 0]
        o_ref[...] = jax.nn.sigmoid(z)

    return pl.pallas_call(
        body,
        grid=(N // BR,),
        in_specs=[
            pl.BlockSpec((BR, DP), lambda i: (i, 0)),
            pl.BlockSpec((BR, DP), lambda i: (i, 0)),
            pl.BlockSpec((BR, 1), lambda i: (i, 0)),
            pl.BlockSpec((1, DP), lambda i: (0, 0)),
            pl.BlockSpec((1, DP), lambda i: (0, 0)),
            pl.BlockSpec((1, 1), lambda i: (0, 0)),
        ],
        out_specs=pl.BlockSpec((BR, 1), lambda i: (i, 0)),
        out_shape=jax.ShapeDtypeStruct((N, 1), jnp.float32),
    )(s_agg, hw, dinv, b, w_out_row, b_out)


def _pad_w(w):
    return jnp.pad(w, ((0, 0), (0, DP - w.shape[1])))


def _pad_sq(w):
    return jnp.pad(w, ((0, DP - w.shape[0]), (0, DP - w.shape[1])))


def _pad_b(b):
    return jnp.pad(b, (0, DP - b.shape[0])).reshape(1, DP)


def kernel(x, edge_index, W_in, b_in, W1, b1, W2, b2, W3, b3, W_out, b_out):
    src = edge_index[0]
    dst = edge_index[1]

    w_in_p = _pad_w(W_in)
    w1_p = _pad_sq(W1)
    w2_p = _pad_sq(W2)
    w3_p = _pad_sq(W3)
    b_in_p = _pad_b(b_in)
    b1_p = _pad_b(b1)
    b2_p = _pad_b(b2)
    b3_p = _pad_b(b3)
    w_out_row = jnp.pad(W_out[:, 0], (0, DP - D_H)).reshape(1, DP)
    b_out_2d = b_out.reshape(1, 1)

    src_p = jnp.pad(src, (0, EPAD - E))
    dst_p = jnp.pad(dst, (0, EPAD - E), constant_values=N)

    degp = _sc_degree(dst)
    hw1 = _tc_encode(x, w_in_p, b_in_p, w1_p)
    g1, dinv = _tc_prep(hw1, degp[0], degp[1])
    s1 = _sc_aggregate(g1, src_p, dst_p)
    g2, hw2 = _tc_combine(s1, hw1, dinv, b1_p, w2_p)
    s2 = _sc_aggregate(g2, src_p, dst_p)
    g3, hw3 = _tc_combine(s2, hw2, dinv, b2_p, w3_p)
    s3 = _sc_aggregate(g3, src_p, dst_p)
    out = _tc_head(s3, hw3, dinv, b3_p, w_out_row, b_out_2d)
    return out.reshape(1, -1)
